# Initial kernel scaffold; baseline (speedup 1.0000x reference)
import functools

import jax
import jax.numpy as jnp
from jax import lax
from jax.experimental import pallas as pl
from jax.experimental.pallas import tpu as pltpu
from jax.experimental.pallas import tpu_sc as plsc

N_NODES = 10000
N_EDGES = 320000
D = 128

NC = 2
NS = 16
NW = NC * NS
E_PER_W = N_EDGES // NW
CHUNK = 125
N_CHUNKS = E_PER_W // CHUNK
ROWS_PER_SUB = N_NODES // NS


def _mlp_block_kernel(x_ref, w_ref, b_ref, o_ref):
    h = jnp.dot(x_ref[...], w_ref[...], preferred_element_type=jnp.float32)
    h = jnp.maximum(h + b_ref[...], 0.0)
    o_ref[...] = jax.nn.sigmoid(h)


def _edge_mlp(edge_attr, W1, b1):
    blk = 4000
    grid = N_EDGES // blk
    return pl.pallas_call(
        _mlp_block_kernel,
        grid=(grid,),
        in_specs=[
            pl.BlockSpec((blk, D), lambda i: (i, 0)),
            pl.BlockSpec((D, D), lambda i: (0, 0)),
            pl.BlockSpec((1, D), lambda i: (0, 0)),
        ],
        out_specs=pl.BlockSpec((blk, D), lambda i: (i, 0)),
        out_shape=jax.ShapeDtypeStruct((N_EDGES, D), jnp.float32),
    )(edge_attr, W1, b1.reshape(1, D))


def _sum_mlp_kernel(p_ref, w_ref, b_ref, o_ref):
    s = p_ref[0] + p_ref[1]
    h = jnp.dot(s, w_ref[...], preferred_element_type=jnp.float32)
    h = jnp.maximum(h + b_ref[...], 0.0)
    o_ref[...] = jax.nn.sigmoid(h)


def _node_mlp(partials, W2, b2):
    blk = 2000
    grid = N_NODES // blk
    return pl.pallas_call(
        _sum_mlp_kernel,
        grid=(grid,),
        in_specs=[
            pl.BlockSpec((NC, blk, D), lambda i: (0, i, 0)),
            pl.BlockSpec((D, D), lambda i: (0, 0)),
            pl.BlockSpec((1, D), lambda i: (0, 0)),
        ],
        out_specs=pl.BlockSpec((blk, D), lambda i: (i, 0)),
        out_shape=jax.ShapeDtypeStruct((N_NODES, D), jnp.float32),
    )(partials, W2, b2.reshape(1, D))


def _sc_scatter_body(h_hbm, col_hbm, zero_hbm, out_hbm, idx_v, rows_v, acc_shared):
    cid = lax.axis_index("c")
    sid = lax.axis_index("s")
    wid = cid * NS + sid

    pltpu.sync_copy(
        zero_hbm.at[pl.ds(sid * ROWS_PER_SUB, ROWS_PER_SUB)],
        acc_shared.at[pl.ds(sid * ROWS_PER_SUB, ROWS_PER_SUB)],
    )
    plsc.subcore_barrier()

    pltpu.sync_copy(col_hbm.at[wid], idx_v)

    base = wid * E_PER_W

    def chunk_step(c, carry):
        pltpu.sync_copy(h_hbm.at[pl.ds(base + c * CHUNK, CHUNK)], rows_v)
        pltpu.sync_copy(rows_v, acc_shared.at[idx_v.at[c]], add=True)
        return carry

    lax.fori_loop(0, N_CHUNKS, chunk_step, 0)
    plsc.subcore_barrier()

    pltpu.sync_copy(
        acc_shared.at[pl.ds(sid * ROWS_PER_SUB, ROWS_PER_SUB)],
        out_hbm.at[cid].at[pl.ds(sid * ROWS_PER_SUB, ROWS_PER_SUB)],
    )


def _sc_scatter(h, col3, zeros):
    mesh = plsc.VectorSubcoreMesh(
        core_axis_name="c", subcore_axis_name="s", num_cores=NC, num_subcores=NS
    )
    f = pl.kernel(
        _sc_scatter_body,
        out_type=jax.ShapeDtypeStruct((NC, N_NODES, D), jnp.float32),
        mesh=mesh,
        scratch_types=[
            pltpu.VMEM((N_CHUNKS, CHUNK), jnp.int32),
            pltpu.VMEM((CHUNK, D), jnp.float32),
            pltpu.VMEM_SHARED((N_NODES, D), jnp.float32),
        ],
    )
    return f(h, col3, zeros)


@jax.jit
def kernel(x, edge_index, edge_attr, u, batch, W1, b1, W2, b2):
    h = _edge_mlp(edge_attr, W1, b1)
    col3 = edge_index[1].astype(jnp.int32).reshape(NW, N_CHUNKS, CHUNK)
    zeros = jnp.zeros((N_NODES, D), jnp.float32)
    partials = _sc_scatter(h, col3, zeros)
    return _node_mlp(partials, W2, b2)


# speedup vs baseline: 3.1754x; 3.1754x over previous
import functools

import jax
import jax.numpy as jnp
from jax import lax
from jax.experimental import pallas as pl
from jax.experimental.pallas import tpu as pltpu
from jax.experimental.pallas import tpu_sc as plsc

N_NODES = 10000
N_EDGES = 320000
D = 128

NC = 2
NS = 16
NW = NC * NS
E_PER_W = N_EDGES // NW
CHUNK = 80
N_CHUNKS = E_PER_W // CHUNK
N_PAD = 10240
ROWS_PER_SUB = N_PAD // NS


def _mlp_block_kernel(x_ref, w_ref, b_ref, o_ref):
    h = jnp.dot(x_ref[...], w_ref[...], preferred_element_type=jnp.float32)
    h = jnp.maximum(h + b_ref[...], 0.0)
    o_ref[...] = jax.nn.sigmoid(h)


def _edge_mlp(edge_attr, W1, b1):
    blk = 4000
    grid = N_EDGES // blk
    return pl.pallas_call(
        _mlp_block_kernel,
        grid=(grid,),
        in_specs=[
            pl.BlockSpec((blk, D), lambda i: (i, 0)),
            pl.BlockSpec((D, D), lambda i: (0, 0)),
            pl.BlockSpec((1, D), lambda i: (0, 0)),
        ],
        out_specs=pl.BlockSpec((blk, D), lambda i: (i, 0)),
        out_shape=jax.ShapeDtypeStruct((N_EDGES, D), jnp.float32),
    )(edge_attr, W1, b1.reshape(1, D))


def _sum_mlp_kernel(p_ref, w_ref, b_ref, o_ref):
    s = p_ref[0] + p_ref[1]
    h = jnp.dot(s, w_ref[...], preferred_element_type=jnp.float32)
    h = jnp.maximum(h + b_ref[...], 0.0)
    o_ref[...] = jax.nn.sigmoid(h)


def _node_mlp(partials, W2, b2):
    blk = 2000
    grid = N_NODES // blk
    return pl.pallas_call(
        _sum_mlp_kernel,
        grid=(grid,),
        in_specs=[
            pl.BlockSpec((NC, blk, D), lambda i: (0, i, 0)),
            pl.BlockSpec((D, D), lambda i: (0, 0)),
            pl.BlockSpec((1, D), lambda i: (0, 0)),
        ],
        out_specs=pl.BlockSpec((blk, D), lambda i: (i, 0)),
        out_shape=jax.ShapeDtypeStruct((N_NODES, D), jnp.float32),
    )(partials, W2, b2.reshape(1, D))


def _sc_scatter_body(h_hbm, col_hbm, zero_hbm, out_hbm, idx_v, rows_v, acc_shared):
    cid = lax.axis_index("c")
    sid = lax.axis_index("s")
    wid = cid * NS + sid

    pltpu.sync_copy(
        zero_hbm,
        acc_shared.at[pl.ds(sid * ROWS_PER_SUB, ROWS_PER_SUB)],
    )
    plsc.subcore_barrier()

    pltpu.sync_copy(col_hbm.at[wid], idx_v)

    base = wid * E_PER_W

    def chunk_step(c, carry):
        pltpu.sync_copy(h_hbm.at[pl.ds(base + c * CHUNK, CHUNK)], rows_v)
        pltpu.sync_copy(rows_v, acc_shared.at[idx_v.at[c]], add=True)
        return carry

    lax.fori_loop(0, N_CHUNKS, chunk_step, 0)
    plsc.subcore_barrier()

    pltpu.sync_copy(
        acc_shared.at[pl.ds(sid * ROWS_PER_SUB, ROWS_PER_SUB)],
        out_hbm.at[cid].at[pl.ds(sid * ROWS_PER_SUB, ROWS_PER_SUB)],
    )


def _sc_scatter(h, col3, zeros):
    mesh = plsc.VectorSubcoreMesh(
        core_axis_name="c", subcore_axis_name="s", num_cores=NC, num_subcores=NS
    )
    f = pl.kernel(
        _sc_scatter_body,
        out_type=jax.ShapeDtypeStruct((NC, N_PAD, D), jnp.float32),
        mesh=mesh,
        scratch_types=[
            pltpu.VMEM((N_CHUNKS, CHUNK), jnp.int32),
            pltpu.VMEM((CHUNK, D), jnp.float32),
            pltpu.VMEM_SHARED((N_PAD, D), jnp.float32),
        ],
    )
    return f(h, col3, zeros)


@jax.jit
def kernel(x, edge_index, edge_attr, u, batch, W1, b1, W2, b2):
    h = _edge_mlp(edge_attr, W1, b1)
    col3 = edge_index[1].astype(jnp.int32).reshape(NW, N_CHUNKS, CHUNK)
    zeros = jnp.zeros((ROWS_PER_SUB, D), jnp.float32)
    partials = _sc_scatter(h, col3, zeros)
    return _node_mlp(partials, W2, b2)


# speedup vs baseline: 3.5981x; 1.1331x over previous
import functools

import jax
import jax.numpy as jnp
from jax import lax
from jax.experimental import pallas as pl
from jax.experimental.pallas import tpu as pltpu
from jax.experimental.pallas import tpu_sc as plsc

N_NODES = 10000
N_EDGES = 320000
D = 128

NC = 2
NS = 16
NW = NC * NS
E_PER_W = N_EDGES // NW
CHUNK = 40
N_CHUNKS = E_PER_W // CHUNK
GROUP = 1
G_ROWS = GROUP * CHUNK
NG = N_CHUNKS // GROUP
N_PAD = 10240
ROWS_PER_SUB = N_PAD // NS


def _mlp_block_kernel(x_ref, w_ref, b_ref, o_ref):
    h = jnp.dot(x_ref[...], w_ref[...], preferred_element_type=jnp.float32)
    h = jnp.maximum(h + b_ref[...], 0.0)
    o_ref[...] = jax.nn.sigmoid(h)


def _edge_mlp(edge_attr, W1, b1):
    blk = 4000
    grid = N_EDGES // blk
    return pl.pallas_call(
        _mlp_block_kernel,
        grid=(grid,),
        in_specs=[
            pl.BlockSpec((blk, D), lambda i: (i, 0)),
            pl.BlockSpec((D, D), lambda i: (0, 0)),
            pl.BlockSpec((1, D), lambda i: (0, 0)),
        ],
        out_specs=pl.BlockSpec((blk, D), lambda i: (i, 0)),
        out_shape=jax.ShapeDtypeStruct((N_EDGES, D), jnp.float32),
    )(edge_attr, W1, b1.reshape(1, D))


def _sum_mlp_kernel(p_ref, w_ref, b_ref, o_ref):
    s = p_ref[0] + p_ref[1]
    h = jnp.dot(s, w_ref[...], preferred_element_type=jnp.float32)
    h = jnp.maximum(h + b_ref[...], 0.0)
    o_ref[...] = jax.nn.sigmoid(h)


def _node_mlp(partials, W2, b2):
    blk = 2000
    grid = N_NODES // blk
    return pl.pallas_call(
        _sum_mlp_kernel,
        grid=(grid,),
        in_specs=[
            pl.BlockSpec((NC, blk, D), lambda i: (0, i, 0)),
            pl.BlockSpec((D, D), lambda i: (0, 0)),
            pl.BlockSpec((1, D), lambda i: (0, 0)),
        ],
        out_specs=pl.BlockSpec((blk, D), lambda i: (i, 0)),
        out_shape=jax.ShapeDtypeStruct((N_NODES, D), jnp.float32),
    )(partials, W2, b2.reshape(1, D))


def _sc_scatter_body(h_hbm, col_hbm, zero_hbm, out_hbm,
                     idx_v, buf0, buf1, acc_shared, gsem0, gsem1):
    cid = lax.axis_index("c")
    sid = lax.axis_index("s")
    wid = cid * NS + sid

    pltpu.sync_copy(
        zero_hbm,
        acc_shared.at[pl.ds(sid * ROWS_PER_SUB, ROWS_PER_SUB)],
    )
    plsc.subcore_barrier()

    pltpu.sync_copy(col_hbm.at[wid], idx_v)

    base = wid * E_PER_W
    bufs = (buf0, buf1)
    sems = (gsem0, gsem1)

    def fire_gather(g, buf, sem):
        pltpu.async_copy(h_hbm.at[pl.ds(base + g * G_ROWS, G_ROWS)], buf, sem)

    def wait_gather(buf, sem):
        pltpu.make_async_copy(h_hbm.at[pl.ds(0, G_ROWS)], buf, sem).wait()

    def scatter_group(g, buf):
        for k in range(GROUP):
            c = g * GROUP + k
            pltpu.sync_copy(
                buf.at[pl.ds(k * CHUNK, CHUNK)],
                acc_shared.at[idx_v.at[c]],
                add=True,
            )

    fire_gather(0, buf0, gsem0)

    def outer(o, carry):
        for b in range(2):
            g = 2 * o + b
            fire_gather(g + 1, bufs[1 - b], sems[1 - b])
            wait_gather(bufs[b], sems[b])
            scatter_group(g, bufs[b])
        return carry

    n_outer = (NG - 1) // 2
    lax.fori_loop(0, n_outer, outer, 0)
    for g in range(2 * n_outer, NG):
        if g + 1 < NG:
            fire_gather(g + 1, bufs[(g + 1) % 2], sems[(g + 1) % 2])
        wait_gather(bufs[g % 2], sems[g % 2])
        scatter_group(g, bufs[g % 2])
    plsc.subcore_barrier()

    pltpu.sync_copy(
        acc_shared.at[pl.ds(sid * ROWS_PER_SUB, ROWS_PER_SUB)],
        out_hbm.at[cid].at[pl.ds(sid * ROWS_PER_SUB, ROWS_PER_SUB)],
    )


def _sc_scatter(h, col3, zeros):
    mesh = plsc.VectorSubcoreMesh(
        core_axis_name="c", subcore_axis_name="s", num_cores=NC, num_subcores=NS
    )
    f = pl.kernel(
        _sc_scatter_body,
        out_type=jax.ShapeDtypeStruct((NC, N_PAD, D), jnp.float32),
        mesh=mesh,
        scratch_types=[
            pltpu.VMEM((N_CHUNKS, CHUNK), jnp.int32),
            pltpu.VMEM((G_ROWS, D), jnp.float32),
            pltpu.VMEM((G_ROWS, D), jnp.float32),
            pltpu.VMEM_SHARED((N_PAD, D), jnp.float32),
            pltpu.SemaphoreType.DMA,
            pltpu.SemaphoreType.DMA,
        ],
    )
    return f(h, col3, zeros)


@jax.jit
def kernel(x, edge_index, edge_attr, u, batch, W1, b1, W2, b2):
    h = _edge_mlp(edge_attr, W1, b1)
    col3 = edge_index[1].astype(jnp.int32).reshape(NW, N_CHUNKS, CHUNK)
    zeros = jnp.zeros((ROWS_PER_SUB, D), jnp.float32)
    partials = _sc_scatter(h, col3, zeros)
    return _node_mlp(partials, W2, b2)


# speedup vs baseline: 3.6876x; 1.0249x over previous
import functools

import jax
import jax.numpy as jnp
from jax import lax
from jax.experimental import pallas as pl
from jax.experimental.pallas import tpu as pltpu
from jax.experimental.pallas import tpu_sc as plsc

N_NODES = 10000
N_EDGES = 320000
D = 128

NC = 2
NS = 16
NW = NC * NS
E_PER_W = N_EDGES // NW
CHUNK = 80
N_CHUNKS = E_PER_W // CHUNK
N_PHASE = 5
PCH = N_CHUNKS // N_PHASE
N_PAD = 10240
ROWS_PER_SUB = N_PAD // NS


def _mlp_block_kernel(x_ref, w_ref, b_ref, o_ref):
    h = jnp.dot(x_ref[...], w_ref[...], preferred_element_type=jnp.float32)
    h = jnp.maximum(h + b_ref[...], 0.0)
    o_ref[...] = jax.nn.sigmoid(h)


def _edge_mlp(edge_attr, W1, b1):
    blk = 4000
    grid = N_EDGES // blk
    return pl.pallas_call(
        _mlp_block_kernel,
        grid=(grid,),
        in_specs=[
            pl.BlockSpec((blk, D), lambda i: (i, 0)),
            pl.BlockSpec((D, D), lambda i: (0, 0)),
            pl.BlockSpec((1, D), lambda i: (0, 0)),
        ],
        out_specs=pl.BlockSpec((blk, D), lambda i: (i, 0)),
        out_shape=jax.ShapeDtypeStruct((N_EDGES, D), jnp.float32),
    )(edge_attr, W1, b1.reshape(1, D))


def _sum_mlp_kernel(p_ref, w_ref, b_ref, o_ref):
    s = p_ref[0] + p_ref[1]
    h = jnp.dot(s, w_ref[...], preferred_element_type=jnp.float32)
    h = jnp.maximum(h + b_ref[...], 0.0)
    o_ref[...] = jax.nn.sigmoid(h)


def _node_mlp(partials, W2, b2):
    blk = 2000
    grid = N_NODES // blk
    return pl.pallas_call(
        _sum_mlp_kernel,
        grid=(grid,),
        in_specs=[
            pl.BlockSpec((NC, blk, D), lambda i: (0, i, 0)),
            pl.BlockSpec((D, D), lambda i: (0, 0)),
            pl.BlockSpec((1, D), lambda i: (0, 0)),
        ],
        out_specs=pl.BlockSpec((blk, D), lambda i: (i, 0)),
        out_shape=jax.ShapeDtypeStruct((N_NODES, D), jnp.float32),
    )(partials, W2, b2.reshape(1, D))


def _sc_scatter_body(h_hbm, col_hbm, zero_hbm, out_hbm,
                     idx0, idx1, buf0, buf1, acc_shared,
                     gsem0, gsem1, ssem0, ssem1):
    cid = lax.axis_index("c")
    sid = lax.axis_index("s")
    wid = cid * NS + sid

    pltpu.sync_copy(
        zero_hbm,
        acc_shared.at[pl.ds(sid * ROWS_PER_SUB, ROWS_PER_SUB)],
    )
    plsc.subcore_barrier()

    base = wid * E_PER_W
    bufs = (buf0, buf1)
    gsems = (gsem0, gsem1)
    ssems = (ssem0, ssem1)
    idxbufs = (idx0, idx1)

    def fire_gather(c, m):
        pltpu.async_copy(h_hbm.at[pl.ds(base + c * CHUNK, CHUNK)], bufs[m], gsems[m])

    def wait_gather(m):
        pltpu.make_async_copy(h_hbm.at[pl.ds(0, CHUNK)], bufs[m], gsems[m]).wait()

    def wait_scatter(m):
        pltpu.make_async_copy(bufs[m], acc_shared.at[pl.ds(0, CHUNK)], ssems[m]).wait()

    def do_chunk(c, m, idxrow, first=False, last=False):
        wait_gather(m)
        pltpu.async_copy(bufs[m], acc_shared.at[idxrow], ssems[m], add=True)
        if not first:
            wait_scatter(1 - m)
        if not last:
            fire_gather(c + 1, 1 - m)

    pltpu.sync_copy(col_hbm.at[wid, 0], idx0)
    fire_gather(0, 0)
    do_chunk(0, 0, idx0.at[0], first=True)

    for p in range(N_PHASE):
        ib = idxbufs[p % 2]
        if p > 0:
            pltpu.sync_copy(col_hbm.at[wid, p], ib)

        if p == 0:
            def inner0(o, carry):
                for b in range(2):
                    j = 1 + 2 * o + b
                    do_chunk(j, (1 + b) % 2, idx0.at[j])
                return carry
            lax.fori_loop(0, PCH // 2, inner0, 0)
        else:
            def inner(o, carry, p=p, ib=ib):
                for b in range(2):
                    j = 2 * o + b
                    do_chunk(p * PCH + j, (p + b) % 2, ib.at[j])
                return carry
            lax.fori_loop(0, PCH // 2, inner, 0)
            j = PCH - 1
            c = p * PCH + j
            do_chunk(c, (p + j) % 2, ib.at[j], last=(c == N_CHUNKS - 1))

    wait_scatter((N_CHUNKS - 1) % 2)
    plsc.subcore_barrier()

    pltpu.sync_copy(
        acc_shared.at[pl.ds(sid * ROWS_PER_SUB, ROWS_PER_SUB)],
        out_hbm.at[cid].at[pl.ds(sid * ROWS_PER_SUB, ROWS_PER_SUB)],
    )


def _sc_scatter(h, col3, zeros):
    mesh = plsc.VectorSubcoreMesh(
        core_axis_name="c", subcore_axis_name="s", num_cores=NC, num_subcores=NS
    )
    f = pl.kernel(
        _sc_scatter_body,
        out_type=jax.ShapeDtypeStruct((NC, N_PAD, D), jnp.float32),
        mesh=mesh,
        scratch_types=[
            pltpu.VMEM((PCH, CHUNK), jnp.int32),
            pltpu.VMEM((PCH, CHUNK), jnp.int32),
            pltpu.VMEM((CHUNK, D), jnp.float32),
            pltpu.VMEM((CHUNK, D), jnp.float32),
            pltpu.VMEM_SHARED((N_PAD, D), jnp.float32),
            pltpu.SemaphoreType.DMA,
            pltpu.SemaphoreType.DMA,
            pltpu.SemaphoreType.DMA,
            pltpu.SemaphoreType.DMA,
        ],
    )
    return f(h, col3, zeros)


@jax.jit
def kernel(x, edge_index, edge_attr, u, batch, W1, b1, W2, b2):
    h = _edge_mlp(edge_attr, W1, b1)
    col3 = edge_index[1].astype(jnp.int32).reshape(NW, N_PHASE, PCH, CHUNK)
    zeros = jnp.zeros((ROWS_PER_SUB, D), jnp.float32)
    partials = _sc_scatter(h, col3, zeros)
    return _node_mlp(partials, W2, b2)
